# Initial kernel scaffold; baseline (speedup 1.0000x reference)
import jax
import jax.numpy as jnp
from jax.experimental import pallas as pl


def kernel(idxs, table):
    raise NotImplementedError("write your pallas kernel here")


# speedup vs baseline: 1.5249x; 1.5249x over previous
import functools

import jax
import jax.numpy as jnp
from jax import lax
from jax.experimental import pallas as pl
from jax.experimental.pallas import tpu as pltpu
from jax.experimental.pallas import tpu_sc as plsc

SEQ_LEN = 8192
EMB_DIM = 1024
NUM_WORKERS = 32
B_PER_W = SEQ_LEN // NUM_WORKERS
CHUNK = 32
NCHUNK = B_PER_W // CHUNK


def _make_lookup():
  mesh = plsc.VectorSubcoreMesh(core_axis_name="c", subcore_axis_name="s")

  @functools.partial(
      pl.kernel,
      mesh=mesh,
      out_type=jax.ShapeDtypeStruct((SEQ_LEN, EMB_DIM), jnp.float32),
      scratch_types=[
          pltpu.VMEM((B_PER_W,), jnp.int32),
          pltpu.VMEM((2, CHUNK, EMB_DIM), jnp.float32),
          pltpu.SemaphoreType.DMA,
          pltpu.SemaphoreType.DMA,
      ],
  )
  def lookup(idx_hbm, table_hbm, out_hbm, idx_v, rows_v, gsem_a, gsem_b):
    wid = lax.axis_index("s") * 2 + lax.axis_index("c")
    base = wid * B_PER_W
    pltpu.sync_copy(idx_hbm.at[pl.ds(base, B_PER_W)], idx_v)

    gsems = (gsem_a, gsem_b)

    def start_gather(c):
      buf = c % 2
      return pltpu.async_copy(
          table_hbm.at[idx_v.at[pl.ds(c * CHUNK, CHUNK)]],
          rows_v.at[buf],
          gsems[buf],
      )

    pending = start_gather(0)
    for c in range(NCHUNK):
      nxt = start_gather(c + 1) if c + 1 < NCHUNK else None
      pending.wait()
      pltpu.sync_copy(rows_v.at[c % 2],
                      out_hbm.at[pl.ds(base + c * CHUNK, CHUNK)])
      pending = nxt

  return lookup


_lookup = _make_lookup()


@jax.jit
def kernel(idxs, table):
  return _lookup(idxs.astype(jnp.int32), table)


# speedup vs baseline: 1.5355x; 1.0070x over previous
import functools

import jax
import jax.numpy as jnp
from jax import lax
from jax.experimental import pallas as pl
from jax.experimental.pallas import tpu as pltpu
from jax.experimental.pallas import tpu_sc as plsc

SEQ_LEN = 8192
EMB_DIM = 1024
NUM_WORKERS = 32
B_PER_W = SEQ_LEN // NUM_WORKERS
CHUNK = 32
NCHUNK = B_PER_W // CHUNK


def _make_lookup():
  mesh = plsc.VectorSubcoreMesh(core_axis_name="c", subcore_axis_name="s")

  @functools.partial(
      pl.kernel,
      mesh=mesh,
      out_type=jax.ShapeDtypeStruct((SEQ_LEN, EMB_DIM), jnp.float32),
      scratch_types=[
          pltpu.VMEM((B_PER_W,), jnp.int32),
          pltpu.VMEM((2, CHUNK, EMB_DIM), jnp.float32),
          pltpu.SemaphoreType.DMA,
          pltpu.SemaphoreType.DMA,
      ],
  )
  def lookup(idx_hbm, table_hbm, out_hbm, idx_v, rows_v, gsem_a, gsem_b):
    wid = lax.axis_index("s") * 2 + lax.axis_index("c")
    base = wid * B_PER_W
    pltpu.sync_copy(idx_hbm.at[pl.ds(base, B_PER_W)], idx_v)

    gsems = (gsem_a, gsem_b)

    def gather_copy(c, buf):
      return pltpu.make_async_copy(
          table_hbm.at[idx_v.at[pl.ds(c * CHUNK, CHUNK)]],
          rows_v.at[buf],
          gsems[buf],
      )

    def write_back(c, buf):
      pltpu.sync_copy(rows_v.at[buf],
                      out_hbm.at[pl.ds(base + c * CHUNK, CHUNK)])

    gather_copy(0, 0).start()

    def body(c, carry):
      gather_copy(c + 1, 1).start()
      gather_copy(c, 0).wait()
      write_back(c, 0)

      @pl.when(c + 2 < NCHUNK)
      def _():
        gather_copy(c + 2, 0).start()

      gather_copy(c + 1, 1).wait()
      write_back(c + 1, 1)
      return carry

    lax.fori_loop(0, NCHUNK // 2, lambda i, carry: body(i * 2, carry), 0,
                  unroll=False)

  return lookup


_lookup = _make_lookup()


@jax.jit
def kernel(idxs, table):
  return _lookup(idxs.astype(jnp.int32), table)
